# baseline (speedup 1.0000x reference)
import dataclasses
import functools

import jax
import jax.numpy as jnp
from jax import lax
from jax.experimental import pallas as pl
from jax.experimental.pallas import tpu as pltpu
from jax.experimental.pallas import tpu_sc as plsc

N_NODES = 10000
D = 128
E = 320000
MIN_NORM = 1e-15
EPS = 1e-7

NC = 2
NS = 16
NW = NC * NS
CHUNK = 128
NCH = 80
E_PAD = CHUNK * NCH * NW
TOT_CHUNKS = E_PAD // CHUNK
N_PAD = 10240
ROWS_PER_SUB = N_PAD // NS
DH = D // 2
SBLK = 16
NBLK = NCH // SBLK
LANES = 16


def _sinh(x):
    return 0.5 * (jnp.exp(x) - jnp.exp(-x))


def _splat16(i):
    return jnp.full((LANES,), i, dtype=jnp.int32)


def _hop_body(xlr_hbm, src_hbm, dst_hbm, w_hbm, out_hbm,
              xs, acc, src_v, dst_v, w_v, rows_a, rows_b, sem_a, sem_b):
    core = lax.axis_index("c")
    sub = lax.axis_index("s")
    wid = core * NS + sub
    zero = jnp.zeros((LANES,), jnp.float32)

    def scale(rows, c):
        @functools.partial(plsc.parallel_loop, 0, CHUNK, unroll=4)
        def _(i):
            wv = plsc.load_gather(w_v, [_splat16(c), _splat16(i)])
            for j in range(DH // LANES):
                sl = (i, pl.ds(j * LANES, LANES))
                rows[sl] = rows[sl] * wv

    for p in range(2):
        my_rows = pl.ds(sub * ROWS_PER_SUB, ROWS_PER_SUB)
        pltpu.sync_copy(xlr_hbm.at[p, 0, my_rows], acc.at[my_rows])
        pltpu.sync_copy(xlr_hbm.at[p, 1, my_rows], xs.at[my_rows])
        for r in range(ROWS_PER_SUB // CHUNK):
            ch = pl.ds(sub * ROWS_PER_SUB + r * CHUNK, CHUNK)
            pltpu.sync_copy(acc.at[ch], rows_a)
            pltpu.sync_copy(xs.at[ch], rows_b)

            @pl.loop(0, CHUNK)
            def _(i):
                for j in range(DH // LANES):
                    sl = (i, pl.ds(j * LANES, LANES))
                    rows_a[sl] = rows_a[sl] + rows_b[sl]

            pltpu.sync_copy(rows_a, xs.at[ch])

        @pl.loop(0, CHUNK)
        def _(i):
            for j in range(DH // LANES):
                rows_a[i, pl.ds(j * LANES, LANES)] = zero

        for r in range(ROWS_PER_SUB // CHUNK):
            pltpu.sync_copy(
                rows_a, acc.at[pl.ds(sub * ROWS_PER_SUB + r * CHUNK, CHUNK)])

        plsc.subcore_barrier()

        for b in range(NBLK):
            base = wid * NCH + b * SBLK
            pltpu.sync_copy(src_hbm.at[pl.ds(base, SBLK)], src_v)
            pltpu.sync_copy(dst_hbm.at[pl.ds(base, SBLK)], dst_v)
            pltpu.sync_copy(w_hbm.at[pl.ds(base, SBLK)], w_v)
            pltpu.async_copy(xs.at[src_v.at[0]], rows_a, sem_a)

            @pl.loop(0, SBLK, step=2)
            def _(c):
                pltpu.async_copy(xs.at[src_v.at[c + 1]], rows_b, sem_b)
                pltpu.make_async_copy(xs.at[src_v.at[c]], rows_a, sem_a).wait()
                scale(rows_a, c)
                pltpu.sync_copy(rows_a, acc.at[dst_v.at[c]], add=True)

                @pl.when(c + 2 < SBLK)
                def _():
                    pltpu.async_copy(xs.at[src_v.at[c + 2]], rows_a, sem_a)

                pltpu.make_async_copy(xs.at[src_v.at[c + 1]], rows_b, sem_b).wait()
                scale(rows_b, c + 1)
                pltpu.sync_copy(rows_b, acc.at[dst_v.at[c + 1]], add=True)

        plsc.subcore_barrier()
        pltpu.sync_copy(
            acc.at[pl.ds(sub * ROWS_PER_SUB, ROWS_PER_SUB)],
            out_hbm.at[p, core, pl.ds(sub * ROWS_PER_SUB, ROWS_PER_SUB)])


def _sc_compiler_params():
    cp = pltpu.CompilerParams()
    if "needs_layout_passes" in pltpu.CompilerParams.__dataclass_fields__:
        cp = dataclasses.replace(cp, needs_layout_passes=False)
    return cp


@jax.jit
def _hop(xlr, src_r, dst_r, w_r):
    mesh = plsc.VectorSubcoreMesh(core_axis_name="c", subcore_axis_name="s")
    f = pl.kernel(
        _hop_body,
        out_type=jax.ShapeDtypeStruct((2, NC, N_PAD, DH), jnp.float32),
        mesh=mesh,
        scratch_types=[
            pltpu.VMEM_SHARED((N_PAD, DH), jnp.float32),
            pltpu.VMEM_SHARED((N_PAD, DH), jnp.float32),
            pltpu.VMEM((SBLK, CHUNK), jnp.int32),
            pltpu.VMEM((SBLK, CHUNK), jnp.int32),
            pltpu.VMEM((SBLK, CHUNK), jnp.float32),
            pltpu.VMEM((CHUNK, DH), jnp.float32),
            pltpu.VMEM((CHUNK, DH), jnp.float32),
            pltpu.SemaphoreType.DMA,
            pltpu.SemaphoreType.DMA,
        ],
        compiler_params=_sc_compiler_params(),
    )
    return f(xlr, src_r, dst_r, w_r)


def _pre_body(emb_ref, out_ref):
    x = emb_ref[...]
    col = lax.broadcasted_iota(jnp.int32, x.shape, 1)
    space = jnp.where(col == 0, 0.0, x)
    sq = jnp.sum(space * space, axis=1, keepdims=True)
    xn = jnp.maximum(jnp.sqrt(sq), MIN_NORM)
    rest = _sinh(xn) * space / xn
    restsq = jnp.sum(rest * rest, axis=1, keepdims=True)
    first = jnp.sqrt(jnp.maximum(1.0 + restsq, EPS))
    yn = jnp.maximum(jnp.sqrt(restsq), MIN_NORM)
    theta = jnp.maximum(first, 1.0 + EPS)
    arc = jnp.log(theta + jnp.sqrt(jnp.maximum(theta * theta - 1.0, 0.0)))
    out = arc * rest / yn
    zero = jnp.zeros_like(out[:, :DH])
    out_ref[0, 0] = out[:, :DH]
    out_ref[1, 0] = out[:, DH:]
    out_ref[0, 1] = zero
    out_ref[1, 1] = zero


@jax.jit
def _pre(emb):
    nb = 8
    rb = N_PAD // nb
    return pl.pallas_call(
        _pre_body,
        grid=(nb,),
        in_specs=[pl.BlockSpec((rb, D), lambda i: (i, 0))],
        out_specs=pl.BlockSpec((2, NC, rb, DH), lambda i: (0, 0, i, 0)),
        out_shape=jax.ShapeDtypeStruct((2, NC, N_PAD, DH), jnp.float32),
    )(emb)


def _post_body(p1_ref, p2_ref, p3_ref, o_ref):
    agg_l = (p1_ref[0, 0] + p1_ref[0, 1] + p2_ref[0, 0] + p2_ref[0, 1]
             + p3_ref[0, 0] + p3_ref[0, 1])
    agg_r = (p1_ref[1, 0] + p1_ref[1, 1] + p2_ref[1, 0] + p2_ref[1, 1]
             + p3_ref[1, 0] + p3_ref[1, 1])
    col = lax.broadcasted_iota(jnp.int32, agg_l.shape, 1)
    space_l = jnp.where(col == 0, 0.0, agg_l)
    sq = (jnp.sum(space_l * space_l, axis=1, keepdims=True)
          + jnp.sum(agg_r * agg_r, axis=1, keepdims=True))
    xn = jnp.maximum(jnp.sqrt(sq), MIN_NORM)
    s = _sinh(xn) / xn
    rest_l = s * space_l
    rest_r = s * agg_r
    restsq = (jnp.sum(rest_l * rest_l, axis=1, keepdims=True)
              + jnp.sum(rest_r * rest_r, axis=1, keepdims=True))
    first = jnp.sqrt(jnp.maximum(1.0 + restsq, EPS))
    o_ref[:, :DH] = jnp.where(col == 0, first, rest_l)
    o_ref[:, DH:] = rest_r


@jax.jit
def _post(p1, p2, p3):
    nb = 8
    rb = N_PAD // nb
    return pl.pallas_call(
        _post_body,
        grid=(nb,),
        in_specs=[pl.BlockSpec((2, NC, rb, DH), lambda i: (0, 0, i, 0)),
                  pl.BlockSpec((2, NC, rb, DH), lambda i: (0, 0, i, 0)),
                  pl.BlockSpec((2, NC, rb, DH), lambda i: (0, 0, i, 0))],
        out_specs=pl.BlockSpec((rb, D), lambda i: (i, 0)),
        out_shape=jax.ShapeDtypeStruct((N_PAD, D), jnp.float32),
    )(p1, p2, p3)


def kernel(emb_weight, edge_index, edge_weight):
    pad = E_PAD - E
    dst = jnp.pad(edge_index[0].astype(jnp.int32), (0, pad))
    src = jnp.pad(edge_index[1].astype(jnp.int32), (0, pad))
    w = jnp.pad(edge_weight.astype(jnp.float32), (0, pad))
    src_r = src.reshape(TOT_CHUNKS, CHUNK)
    dst_r = dst.reshape(TOT_CHUNKS, CHUNK)
    w_r = w.reshape(TOT_CHUNKS, CHUNK)
    emb_p = jnp.pad(emb_weight, ((0, N_PAD - N_NODES), (0, 0)))

    x0 = _pre(emb_p)
    p1 = _hop(x0, src_r, dst_r, w_r)
    p2 = _hop(p1, src_r, dst_r, w_r)
    p3 = _hop(p2, src_r, dst_r, w_r)
    return _post(p1, p2, p3)[:N_NODES]


# speedup vs baseline: 1.3091x; 1.3091x over previous
import dataclasses
import functools

import jax
import jax.numpy as jnp
from jax import lax
from jax.experimental import pallas as pl
from jax.experimental.pallas import tpu as pltpu
from jax.experimental.pallas import tpu_sc as plsc

N_NODES = 10000
D = 128
E = 320000
MIN_NORM = 1e-15
EPS = 1e-7

NC = 2
NS = 16
NW = NC * NS
CHUNK = 128
NCH = 80
E_PAD = CHUNK * NCH * NW
TOT_CHUNKS = E_PAD // CHUNK
N_PAD = 10240
ROWS_PER_SUB = N_PAD // NS
DH = D // 2
SBLK = 40
NBLK = NCH // SBLK
LANES = 16


def _sinh(x):
    return 0.5 * (jnp.exp(x) - jnp.exp(-x))


def _splat16(i):
    return jnp.full((LANES,), i, dtype=jnp.int32)


def _hop_body(xlr_hbm, src_hbm, dst_hbm, w_hbm, out_hbm,
              xs, acc, src_v, dst_v, w_v, rows_a, rows_b, sem_a, sem_b):
    core = lax.axis_index("c")
    sub = lax.axis_index("s")
    wid = core * NS + sub
    zero = jnp.zeros((LANES,), jnp.float32)

    def scale(rows, c):
        @functools.partial(plsc.parallel_loop, 0, CHUNK, unroll=4)
        def _(i):
            wv = plsc.load_gather(w_v, [_splat16(c), _splat16(i)])
            for j in range(DH // LANES):
                sl = (i, pl.ds(j * LANES, LANES))
                rows[sl] = rows[sl] * wv

    for p in range(2):
        pltpu.sync_copy(xlr_hbm.at[p, pl.ds(sub * ROWS_PER_SUB, ROWS_PER_SUB)],
                        xs.at[pl.ds(sub * ROWS_PER_SUB, ROWS_PER_SUB)])

        @pl.loop(0, CHUNK)
        def _(i):
            for j in range(DH // LANES):
                rows_a[i, pl.ds(j * LANES, LANES)] = zero

        for r in range(ROWS_PER_SUB // CHUNK):
            pltpu.sync_copy(
                rows_a, acc.at[pl.ds(sub * ROWS_PER_SUB + r * CHUNK, CHUNK)])

        plsc.subcore_barrier()

        for b in range(NBLK):
            base = wid * NCH + b * SBLK
            pltpu.sync_copy(src_hbm.at[pl.ds(base, SBLK)], src_v)
            pltpu.sync_copy(dst_hbm.at[pl.ds(base, SBLK)], dst_v)
            pltpu.sync_copy(w_hbm.at[pl.ds(base, SBLK)], w_v)
            pltpu.async_copy(xs.at[src_v.at[0]], rows_a, sem_a)

            @pl.loop(0, SBLK, step=2)
            def _(c):
                pltpu.async_copy(xs.at[src_v.at[c + 1]], rows_b, sem_b)
                pltpu.make_async_copy(xs.at[src_v.at[c]], rows_a, sem_a).wait()
                scale(rows_a, c)
                pltpu.sync_copy(rows_a, acc.at[dst_v.at[c]], add=True)

                @pl.when(c + 2 < SBLK)
                def _():
                    pltpu.async_copy(xs.at[src_v.at[c + 2]], rows_a, sem_a)

                pltpu.make_async_copy(xs.at[src_v.at[c + 1]], rows_b, sem_b).wait()
                scale(rows_b, c + 1)
                pltpu.sync_copy(rows_b, acc.at[dst_v.at[c + 1]], add=True)

        plsc.subcore_barrier()
        pltpu.sync_copy(
            acc.at[pl.ds(sub * ROWS_PER_SUB, ROWS_PER_SUB)],
            out_hbm.at[p, core, pl.ds(sub * ROWS_PER_SUB, ROWS_PER_SUB)])


def _sc_compiler_params():
    cp = pltpu.CompilerParams()
    if "needs_layout_passes" in pltpu.CompilerParams.__dataclass_fields__:
        cp = dataclasses.replace(cp, needs_layout_passes=False)
    return cp


@jax.jit
def _hop(xlr, src_r, dst_r, w_r):
    mesh = plsc.VectorSubcoreMesh(core_axis_name="c", subcore_axis_name="s")
    f = pl.kernel(
        _hop_body,
        out_type=jax.ShapeDtypeStruct((2, NC, N_PAD, DH), jnp.float32),
        mesh=mesh,
        scratch_types=[
            pltpu.VMEM_SHARED((N_PAD, DH), jnp.float32),
            pltpu.VMEM_SHARED((N_PAD, DH), jnp.float32),
            pltpu.VMEM((SBLK, CHUNK), jnp.int32),
            pltpu.VMEM((SBLK, CHUNK), jnp.int32),
            pltpu.VMEM((SBLK, CHUNK), jnp.float32),
            pltpu.VMEM((CHUNK, DH), jnp.float32),
            pltpu.VMEM((CHUNK, DH), jnp.float32),
            pltpu.SemaphoreType.DMA,
            pltpu.SemaphoreType.DMA,
        ],
        compiler_params=_sc_compiler_params(),
    )
    return f(xlr, src_r, dst_r, w_r)


def _pre_body(emb_ref, out_ref):
    x = emb_ref[...]
    col = lax.broadcasted_iota(jnp.int32, x.shape, 1)
    space = jnp.where(col == 0, 0.0, x)
    sq = jnp.sum(space * space, axis=1, keepdims=True)
    xn = jnp.maximum(jnp.sqrt(sq), MIN_NORM)
    rest = _sinh(xn) * space / xn
    restsq = jnp.sum(rest * rest, axis=1, keepdims=True)
    first = jnp.sqrt(jnp.maximum(1.0 + restsq, EPS))
    yn = jnp.maximum(jnp.sqrt(restsq), MIN_NORM)
    theta = jnp.maximum(first, 1.0 + EPS)
    arc = jnp.log(theta + jnp.sqrt(jnp.maximum(theta * theta - 1.0, 0.0)))
    out = arc * rest / yn
    out_ref[0] = out[:, :DH]
    out_ref[1] = out[:, DH:]


@jax.jit
def _pre(emb):
    nb = 8
    rb = N_PAD // nb
    return pl.pallas_call(
        _pre_body,
        grid=(nb,),
        in_specs=[pl.BlockSpec((rb, D), lambda i: (i, 0))],
        out_specs=pl.BlockSpec((2, rb, DH), lambda i: (0, i, 0)),
        out_shape=jax.ShapeDtypeStruct((2, N_PAD, DH), jnp.float32),
    )(emb)


def _combine_body(p_ref, o_ref):
    o_ref[0] = p_ref[0, 0] + p_ref[0, 1]
    o_ref[1] = p_ref[1, 0] + p_ref[1, 1]


@jax.jit
def _combine(p):
    nb = 8
    rb = N_PAD // nb
    return pl.pallas_call(
        _combine_body,
        grid=(nb,),
        in_specs=[pl.BlockSpec((2, NC, rb, DH), lambda i: (0, 0, i, 0))],
        out_specs=pl.BlockSpec((2, rb, DH), lambda i: (0, i, 0)),
        out_shape=jax.ShapeDtypeStruct((2, N_PAD, DH), jnp.float32),
    )(p)


def _post_body(x1_ref, x2_ref, p3_ref, o_ref):
    agg_l = x1_ref[0] + x2_ref[0] + p3_ref[0, 0] + p3_ref[0, 1]
    agg_r = x1_ref[1] + x2_ref[1] + p3_ref[1, 0] + p3_ref[1, 1]
    col = lax.broadcasted_iota(jnp.int32, agg_l.shape, 1)
    space_l = jnp.where(col == 0, 0.0, agg_l)
    sq = (jnp.sum(space_l * space_l, axis=1, keepdims=True)
          + jnp.sum(agg_r * agg_r, axis=1, keepdims=True))
    xn = jnp.maximum(jnp.sqrt(sq), MIN_NORM)
    s = _sinh(xn) / xn
    rest_l = s * space_l
    rest_r = s * agg_r
    restsq = (jnp.sum(rest_l * rest_l, axis=1, keepdims=True)
              + jnp.sum(rest_r * rest_r, axis=1, keepdims=True))
    first = jnp.sqrt(jnp.maximum(1.0 + restsq, EPS))
    o_ref[:, :DH] = jnp.where(col == 0, first, rest_l)
    o_ref[:, DH:] = rest_r


@jax.jit
def _post(x1, x2, p3):
    nb = 8
    rb = N_PAD // nb
    return pl.pallas_call(
        _post_body,
        grid=(nb,),
        in_specs=[pl.BlockSpec((2, rb, DH), lambda i: (0, i, 0)),
                  pl.BlockSpec((2, rb, DH), lambda i: (0, i, 0)),
                  pl.BlockSpec((2, NC, rb, DH), lambda i: (0, 0, i, 0))],
        out_specs=pl.BlockSpec((rb, D), lambda i: (i, 0)),
        out_shape=jax.ShapeDtypeStruct((N_PAD, D), jnp.float32),
    )(x1, x2, p3)


def kernel(emb_weight, edge_index, edge_weight):
    pad = E_PAD - E
    dst = jnp.pad(edge_index[0].astype(jnp.int32), (0, pad))
    src = jnp.pad(edge_index[1].astype(jnp.int32), (0, pad))
    w = jnp.pad(edge_weight.astype(jnp.float32), (0, pad))
    src_r = src.reshape(TOT_CHUNKS, CHUNK)
    dst_r = dst.reshape(TOT_CHUNKS, CHUNK)
    w_r = w.reshape(TOT_CHUNKS, CHUNK)
    emb_p = jnp.pad(emb_weight, ((0, N_PAD - N_NODES), (0, 0)))

    x0 = _pre(emb_p)
    p1 = _hop(x0, src_r, dst_r, w_r)
    x1 = _combine(p1)
    p2 = _hop(x1, src_r, dst_r, w_r)
    x2 = _combine(p2)
    p3 = _hop(x2, src_r, dst_r, w_r)
    return _post(x1, x2, p3)[:N_NODES]
